# baseline (speedup 1.0000x reference)
import functools

import jax
import jax.numpy as jnp
from jax import lax
from jax.experimental import pallas as pl
from jax.experimental.pallas import tpu as pltpu
from jax.experimental.pallas import tpu_sc as plsc

NUM_FIELDS = 26
VOCAB = 100000
EMBED = 32
BATCH = 4096
SCAL_DIM = 13
OUT_DIM = SCAL_DIM + NUM_FIELDS * EMBED
NC, NS = 2, 16
NW = NC * NS
LANES = 16
UNROLL = 8


def _lookup_t(tab_t, cat_t, scal_t):
    mesh = plsc.VectorSubcoreMesh(core_axis_name="c", subcore_axis_name="s")

    @functools.partial(
        pl.kernel,
        mesh=mesh,
        compiler_params=pltpu.CompilerParams(needs_layout_passes=False),
        out_type=jax.ShapeDtypeStruct((OUT_DIM, BATCH), jnp.float32),
        scratch_types=[
            pltpu.VMEM((BATCH,), jnp.int32),
            pltpu.VMEM((BATCH,), jnp.int32),
            pltpu.VMEM((VOCAB,), jnp.float32),
            pltpu.VMEM((BATCH,), jnp.float32),
            pltpu.VMEM((BATCH,), jnp.float32),
            pltpu.SemaphoreType.DMA,
            pltpu.SemaphoreType.DMA,
        ],
    )
    def lookup_k(tab_hbm, idx_hbm, scal_hbm, out_hbm,
                 idx_v0, idx_v1, row_v, out_v0, out_v1, sem_idx, sem_out):
        wid = lax.axis_index("s") * NC + lax.axis_index("c")

        @pl.when(wid < SCAL_DIM)
        def _():
            pltpu.sync_copy(scal_hbm.at[wid], out_v0)
            pltpu.sync_copy(out_v0, out_hbm.at[wid])

        pltpu.async_copy(idx_hbm.at[0], idx_v0, sem_idx).wait()
        pltpu.async_copy(idx_hbm.at[1], idx_v1, sem_idx)

        def do_field(f, t, par, idx_s, out_s):
            pltpu.sync_copy(tab_hbm.at[f, wid], row_v)

            @pl.when(t + par >= 1)
            def _():
                pltpu.make_async_copy(idx_hbm.at[f], idx_s, sem_idx).wait()

            @pl.when(t >= 1)
            def _():
                pltpu.make_async_copy(
                    out_s, out_hbm.at[SCAL_DIM + EMBED * (f - 2) + wid], sem_out
                ).wait()

            @pl.loop(0, BATCH // (LANES * UNROLL))
            def _(k):
                for j in range(UNROLL):
                    o = (k * UNROLL + j) * LANES
                    iv = idx_s[pl.ds(o, LANES)]
                    out_s[pl.ds(o, LANES)] = plsc.load_gather(row_v, [iv])

            pltpu.async_copy(
                out_s, out_hbm.at[SCAL_DIM + EMBED * f + wid], sem_out
            )

            @pl.when(f + 2 < NUM_FIELDS)
            def _():
                pltpu.async_copy(idx_hbm.at[f + 2], idx_s, sem_idx)

        @pl.loop(0, NUM_FIELDS // 2)
        def _(t):
            do_field(2 * t, t, 0, idx_v0, out_v0)
            do_field(2 * t + 1, t, 1, idx_v1, out_v1)

        pltpu.make_async_copy(
            out_v0, out_hbm.at[SCAL_DIM + EMBED * (NUM_FIELDS - 2) + wid],
            sem_out,
        ).wait()
        pltpu.make_async_copy(
            out_v1, out_hbm.at[SCAL_DIM + EMBED * (NUM_FIELDS - 1) + wid],
            sem_out,
        ).wait()

    return lookup_k(tab_t, cat_t, scal_t)


def kernel(scal_feat, cat_feat, tables):
    tab_t = jnp.swapaxes(tables, 1, 2)
    cat_t = cat_feat.T
    scal_t = scal_feat.T
    out_t = _lookup_t(tab_t, cat_t, scal_t)
    return out_t.T


# speedup vs baseline: 1.0334x; 1.0334x over previous
import functools

import jax
import jax.numpy as jnp
from jax import lax
from jax.experimental import pallas as pl
from jax.experimental.pallas import tpu as pltpu
from jax.experimental.pallas import tpu_sc as plsc

NUM_FIELDS = 26
VOCAB = 100000
EMBED = 32
BATCH = 4096
SCAL_DIM = 13
OUT_DIM = SCAL_DIM + NUM_FIELDS * EMBED
NC, NS = 2, 16
NW = NC * NS
LANES = 16
UNROLL = 8


def _lookup_t(tab_t, cat_t, scal_t):
    mesh = plsc.VectorSubcoreMesh(core_axis_name="c", subcore_axis_name="s")

    @functools.partial(
        pl.kernel,
        mesh=mesh,
        compiler_params=pltpu.CompilerParams(needs_layout_passes=False),
        out_type=jax.ShapeDtypeStruct((OUT_DIM, BATCH), jnp.float32),
        scratch_types=[
            pltpu.VMEM((BATCH,), jnp.int32),
            pltpu.VMEM((BATCH,), jnp.int32),
            pltpu.VMEM((VOCAB,), jnp.float32),
            pltpu.VMEM((BATCH,), jnp.float32),
            pltpu.VMEM((BATCH,), jnp.float32),
            pltpu.SemaphoreType.DMA,
            pltpu.SemaphoreType.DMA,
            pltpu.SemaphoreType.DMA,
            pltpu.SemaphoreType.DMA,
        ],
    )
    def lookup_k(tab_hbm, idx_hbm, scal_hbm, out_hbm,
                 idx_v0, idx_v1, row_v, out_v0, out_v1,
                 sem_idx0, sem_idx1, sem_out0, sem_out1):
        wid = lax.axis_index("s") * NC + lax.axis_index("c")

        @pl.when(wid < SCAL_DIM)
        def _():
            pltpu.sync_copy(scal_hbm.at[wid], out_v0)
            pltpu.sync_copy(out_v0, out_hbm.at[wid])

        pltpu.async_copy(idx_hbm.at[0], idx_v0, sem_idx0).wait()
        pltpu.async_copy(idx_hbm.at[1], idx_v1, sem_idx1)

        def do_field(f, t, par, idx_s, out_s, sem_idx, sem_out):
            pltpu.sync_copy(tab_hbm.at[f, wid], row_v)

            @pl.when(t + par >= 1)
            def _():
                pltpu.make_async_copy(idx_hbm.at[f], idx_s, sem_idx).wait()

            @pl.when(t >= 1)
            def _():
                pltpu.make_async_copy(
                    out_s, out_hbm.at[SCAL_DIM + EMBED * (f - 2) + wid], sem_out
                ).wait()

            @pl.loop(0, BATCH // (LANES * UNROLL))
            def _(k):
                for j in range(UNROLL):
                    o = (k * UNROLL + j) * LANES
                    iv = idx_s[pl.ds(o, LANES)]
                    out_s[pl.ds(o, LANES)] = plsc.load_gather(row_v, [iv])

            pltpu.async_copy(
                out_s, out_hbm.at[SCAL_DIM + EMBED * f + wid], sem_out
            )

            @pl.when(f + 2 < NUM_FIELDS)
            def _():
                pltpu.async_copy(idx_hbm.at[f + 2], idx_s, sem_idx)

        @pl.loop(0, NUM_FIELDS // 2)
        def _(t):
            do_field(2 * t, t, 0, idx_v0, out_v0, sem_idx0, sem_out0)
            do_field(2 * t + 1, t, 1, idx_v1, out_v1, sem_idx1, sem_out1)

        pltpu.make_async_copy(
            out_v0, out_hbm.at[SCAL_DIM + EMBED * (NUM_FIELDS - 2) + wid],
            sem_out0,
        ).wait()
        pltpu.make_async_copy(
            out_v1, out_hbm.at[SCAL_DIM + EMBED * (NUM_FIELDS - 1) + wid],
            sem_out1,
        ).wait()

    return lookup_k(tab_t, cat_t, scal_t)


def kernel(scal_feat, cat_feat, tables):
    tab_t = jnp.swapaxes(tables, 1, 2)
    cat_t = cat_feat.T
    scal_t = scal_feat.T
    out_t = _lookup_t(tab_t, cat_t, scal_t)
    return out_t.T
